# baseline (speedup 1.0000x reference)
import functools

import jax
import jax.numpy as jnp
from jax import lax
from jax.experimental import pallas as pl
from jax.experimental.pallas import tpu as pltpu
from jax.experimental.pallas import tpu_sc as plsc

NC = 2
NS = 16
NW = NC * NS
D = 256
CAP = 2048
LCH = 128
RS_PAD = 10240
GPAD = 192


def _sread(ref, idx):
    return ref[pl.ds(idx, 16)][0]


def _hist_body(n_nodes, n_edges, src_hbm, hist_hbm, hist_v, src_v):
    ept = n_edges // NW
    wid = lax.axis_index("s") * NC + lax.axis_index("c")
    fb = (wid * ept // 8) * 8
    sh = wid * ept - fb
    iota = lax.iota(jnp.int32, 16)
    ones = jnp.ones((16,), jnp.int32)

    def zero(z, _):
        hist_v[pl.ds(z * 16, 16)] = jnp.zeros((16,), jnp.int32)
        return 0

    lax.fori_loop(0, n_nodes // 16, zero, 0)
    pltpu.sync_copy(src_hbm.at[pl.ds(fb, ept + 16)], src_v)

    def chunk(c, _):
        e = c * 16 + iota
        s16 = plsc.load_gather(src_v, [sh + e])
        plsc.addupdate_scatter(hist_v, [s16], ones, mask=e < ept)
        return 0

    lax.fori_loop(0, (ept + 15) // 16, chunk, 0)
    pltpu.sync_copy(hist_v, hist_hbm.at[wid])


def _place_body(n_nodes, n_edges, src_hbm, dstv_hbm, tb_hbm, grp_hbm,
                off_v, src_v, dstv_v, pos2_v, val2_v, tmp_v, sem):
    ept = n_edges // NW
    nrow = (ept + 16 + 127) // 128
    wid = lax.axis_index("s") * NC + lax.axis_index("c")
    fb = (wid * ept // 8) * 8
    sh = wid * ept - fb
    iota = lax.iota(jnp.int32, 16)
    ones = jnp.ones((16,), jnp.int32)

    pltpu.sync_copy(tb_hbm.at[wid], off_v)
    pltpu.sync_copy(src_hbm.at[pl.ds(fb, ept + 16)], src_v)
    pltpu.sync_copy(dstv_hbm.at[pl.ds(fb, ept + 16)], dstv_v)

    tail0 = ((ept + 15) // 16) * 16
    for q in range((nrow * 128 - tail0 + 15) // 16):
        p = tail0 + q * 16
        pos2_v[p >> 7, pl.ds(p & 127, 16)] = n_edges + iota

    def chunk(c, _):
        e = c * 16 + iota
        valid = e < ept
        s16 = plsc.load_gather(src_v, [sh + e])
        d16 = plsc.load_gather(dstv_v, [sh + e])
        base = plsc.load_gather(off_v, [s16])
        sk, _sv = plsc.sort_key_val(s16, iota)
        tmp_v[pl.ds(0, 16)] = sk
        ev = plsc.load_gather(tmp_v, [jnp.maximum(iota - 1, 0)])
        p = plsc.all_reduce_population_count((ev == sk) & (iota >= 1))
        p = p if p.ndim == 0 else p[0]

        def do_rank(_):
            rank = jnp.zeros((16,), jnp.int32)
            for q in range(1, 16):
                evq = plsc.load_gather(src_v, [sh + jnp.maximum(e - q, 0)])
                rank = rank + jnp.where((evq == s16) & (iota >= q), 1, 0)
            return rank

        rank = lax.cond(p > 0, do_rank,
                        lambda _: jnp.zeros((16,), jnp.int32), 0)
        pos = jnp.where(valid, base + rank, n_edges + iota)
        plsc.store_scatter(pos2_v, [e >> 7, e & 127], pos)
        plsc.store_scatter(val2_v, [e >> 7, e & 127], d16)
        plsc.addupdate_scatter(off_v, [s16], ones, mask=valid)
        return 0

    lax.fori_loop(0, (ept + 15) // 16, chunk, 0)
    handles = [pltpu.async_copy(val2_v.at[k], grp_hbm.at[pos2_v.at[k]], sem)
               for k in range(nrow)]
    for h in handles:
        h.wait()


def _sc_body(n_targets, x_hbm, dst_hbm, rs_hbm, ti_hbm, tj_hbm, w1_hbm, w2_hbm,
             out_hbm, rs_v, ib_v, jb_v, xia_v, xja_v, lia_v, lja_v, xv_v,
             w1_v, w2_v, ti_v, tj_v, out_v, sem):
    ntpw = n_targets // NW
    wid = lax.axis_index("s") * NC + lax.axis_index("c")
    base = wid * ntpw
    pltpu.sync_copy(rs_hbm, rs_v)
    pltpu.sync_copy(ti_hbm.at[pl.ds(base, ntpw)], ti_v.at[pl.ds(0, ntpw)])
    pltpu.sync_copy(tj_hbm.at[pl.ds(base, ntpw)], tj_v.at[pl.ds(0, ntpw)])
    pltpu.sync_copy(w1_hbm, w1_v)
    pltpu.sync_copy(w2_hbm, w2_v)
    iota = lax.iota(jnp.int32, 16)

    def prefetch(t, _):
        i = _sread(ti_v, t)
        j = _sread(tj_v, t)
        fbi = (_sread(rs_v, i) // 8) * 8
        fbj = (_sread(rs_v, j) // 8) * 8
        pltpu.async_copy(x_hbm.at[i], xia_v.at[pl.ds(t * D, D)], sem)
        pltpu.async_copy(x_hbm.at[j], xja_v.at[pl.ds(t * D, D)], sem)
        pltpu.async_copy(dst_hbm.at[pl.ds(fbi, LCH)],
                         lia_v.at[pl.ds(t * LCH, LCH)], sem)
        pltpu.async_copy(dst_hbm.at[pl.ds(fbj, LCH)],
                         lja_v.at[pl.ds(t * LCH, LCH)], sem)
        return 0

    lax.fori_loop(0, ntpw, prefetch, 0)

    def drain(t, _):
        pltpu.make_async_copy(x_hbm.at[0], xia_v.at[pl.ds(t * D, D)], sem).wait()
        pltpu.make_async_copy(x_hbm.at[0], xja_v.at[pl.ds(t * D, D)], sem).wait()
        pltpu.make_async_copy(dst_hbm.at[pl.ds(0, LCH)],
                              lia_v.at[pl.ds(t * LCH, LCH)], sem).wait()
        pltpu.make_async_copy(dst_hbm.at[pl.ds(0, LCH)],
                              lja_v.at[pl.ds(t * LCH, LCH)], sem).wait()
        return 0

    lax.fori_loop(0, ntpw, drain, 0)

    def _intersect(acc, jref, jbase, shj, kj, iref, ibase, shi, ki):
        nchi = (ki + 15) >> 4

        def per_chunk(c, acc):
            gpos = c * 16 + iota
            jv = plsc.load_gather(jref, [jbase + shj + gpos])
            focc = gpos < kj
            for q in range(1, 16):
                ev = plsc.load_gather(jref,
                                      [jbase + shj + jnp.maximum(gpos - q, 0)])
                focc &= ~((ev == jv) & (iota >= q))

            def earlier(c2, focc):
                for q in range(16):
                    ep = c2 * 16 + ((iota + q) & 15)
                    ev = plsc.load_gather(jref, [jbase + shj + ep])
                    focc &= ~((ev == jv) & (ep < kj))
                return focc

            focc = lax.fori_loop(0, c, earlier, focc)

            member = iota < 0

            def ichunk(c2, member):
                for q in range(16):
                    ip = c2 * 16 + ((iota + q) & 15)
                    iv = plsc.load_gather(iref, [ibase + shi + ip])
                    member |= (iv == jv) & (ip < ki)
                return member

            member = lax.fori_loop(0, nchi, ichunk, member)
            member &= focc

            def m_cond(st):
                m, _ = st
                p = plsc.all_reduce_population_count(m)
                p = p if p.ndim == 0 else p[0]
                return p > 0

            def m_body(st):
                m, a = st
                r = plsc.all_reduce_ffs(m)
                l = r if r.ndim == 0 else r[0]
                v = _sread(jref, jbase + shj + c * 16 + l)
                pltpu.sync_copy(x_hbm.at[v], xv_v)
                for cc in range(D // 16):
                    sl = pl.ds(cc * 16, 16)
                    a = a + xv_v[sl] * w2_v[sl]
                return m & (iota != l), a

            member, acc = lax.while_loop(m_cond, m_body, (member, acc))
            return acc

        return lax.fori_loop(0, (kj + 15) >> 4, per_chunk, acc)

    def per_target(t, ovec):
        i = _sread(ti_v, t)
        j = _sread(tj_v, t)
        rsi = _sread(rs_v, i)
        rsj = _sread(rs_v, j)
        ki = _sread(rs_v, i + 1) - rsi
        kj = _sread(rs_v, j + 1) - rsj
        shi = rsi & 7
        shj = rsj & 7

        acc = jnp.zeros((16,), jnp.float32)
        for cc in range(D // 16):
            a_sl = pl.ds(t * D + cc * 16, 16)
            w_sl = pl.ds(cc * 16, 16)
            acc = acc + xia_v[a_sl] * xja_v[a_sl] * w1_v[w_sl]

        def fast(acc):
            return _intersect(acc, lja_v, t * LCH, shj, kj,
                              lia_v, t * LCH, shi, ki)

        def slow(acc):
            kic = jnp.minimum(ki, CAP)
            kjc = jnp.minimum(kj, CAP)

            fbi = (rsi // 8) * 8
            fbj = (rsj // 8) * 8

            def cp_i(c, _):
                pltpu.sync_copy(dst_hbm.at[pl.ds(fbi + c * LCH, LCH)],
                                ib_v.at[pl.ds(c * LCH, LCH)])
                return 0

            def cp_j(c, _):
                pltpu.sync_copy(dst_hbm.at[pl.ds(fbj + c * LCH, LCH)],
                                jb_v.at[pl.ds(c * LCH, LCH)])
                return 0

            lax.fori_loop(0, (kic + shi + LCH - 1) >> 7, cp_i, 0)
            lax.fori_loop(0, (kjc + shj + LCH - 1) >> 7, cp_j, 0)
            return _intersect(acc, jb_v, 0, shj, kjc, ib_v, 0, shi, kic)

        overflow = (shi + ki > LCH) | (shj + kj > LCH)
        acc = lax.cond(overflow, slow, fast, acc)

        for sh in (8, 4, 2, 1):
            xv_v[pl.ds(0, 16)] = acc
            acc = acc + plsc.load_gather(xv_v, [iota ^ sh])
        return jnp.where(iota == (t & 15), acc, ovec)

    def per_group(g, _):
        ovec = lax.fori_loop(g * 16, g * 16 + 16, per_target,
                             jnp.zeros((16,), jnp.float32))
        out_v[pl.ds(g * 16, 16)] = ovec
        return 0

    lax.fori_loop(0, ntpw // 16, per_group, 0)
    pltpu.sync_copy(out_v, out_hbm.at[pl.ds(base, ntpw)])


@functools.partial(jax.jit, static_argnames=("n_nodes", "n_edges", "n_targets"))
def _ncn_sc(x, adj, ti, tj, w1, w2, n_nodes, n_edges, n_targets):
    ntpw = n_targets // NW
    ept = n_edges // NW
    nrow = (ept + 16 + 127) // 128
    mesh = plsc.VectorSubcoreMesh(core_axis_name="c", subcore_axis_name="s")
    cp = pltpu.CompilerParams(needs_layout_passes=False)

    src_pad = jnp.concatenate([adj[0], jnp.zeros((64,), jnp.int32)])
    dstv_pad = jnp.concatenate([adj[1], jnp.zeros((64,), jnp.int32)])

    hist = pl.kernel(
        functools.partial(_hist_body, n_nodes, n_edges),
        out_type=jax.ShapeDtypeStruct((NW, n_nodes), jnp.int32),
        mesh=mesh,
        scratch_types=[
            pltpu.VMEM((n_nodes,), jnp.int32),
            pltpu.VMEM((ept + 16,), jnp.int32),
        ],
        compiler_params=cp,
    )(src_pad)

    counts = hist.sum(axis=0, dtype=jnp.int32)
    row_start = jnp.concatenate(
        [jnp.zeros((1,), jnp.int32), jnp.cumsum(counts, dtype=jnp.int32)])
    rs_pad = jnp.concatenate(
        [row_start, jnp.full((RS_PAD - n_nodes - 1,), n_edges, jnp.int32)])
    tile_base = row_start[:n_nodes][None, :] + (
        jnp.cumsum(hist, axis=0, dtype=jnp.int32) - hist)

    grouped = pl.kernel(
        functools.partial(_place_body, n_nodes, n_edges),
        out_type=jax.ShapeDtypeStruct((n_edges + GPAD,), jnp.int32),
        mesh=mesh,
        scratch_types=[
            pltpu.VMEM((n_nodes,), jnp.int32),
            pltpu.VMEM((ept + 16,), jnp.int32),
            pltpu.VMEM((ept + 16,), jnp.int32),
            pltpu.VMEM((nrow, 128), jnp.int32),
            pltpu.VMEM((nrow, 128), jnp.int32),
            pltpu.VMEM((16,), jnp.int32),
            pltpu.SemaphoreType.DMA,
        ],
        compiler_params=cp,
    )(src_pad, dstv_pad, tile_base)

    out = pl.kernel(
        functools.partial(_sc_body, n_targets),
        out_type=jax.ShapeDtypeStruct((n_targets,), jnp.float32),
        mesh=mesh,
        scratch_types=[
            pltpu.VMEM((RS_PAD,), jnp.int32),
            pltpu.VMEM((CAP + LCH,), jnp.int32),
            pltpu.VMEM((CAP + LCH,), jnp.int32),
            pltpu.VMEM((ntpw * D + 16,), jnp.float32),
            pltpu.VMEM((ntpw * D + 16,), jnp.float32),
            pltpu.VMEM((ntpw * LCH + 32,), jnp.int32),
            pltpu.VMEM((ntpw * LCH + 32,), jnp.int32),
            pltpu.VMEM((D,), jnp.float32),
            pltpu.VMEM((D,), jnp.float32),
            pltpu.VMEM((D,), jnp.float32),
            pltpu.VMEM((ntpw + 16,), jnp.int32),
            pltpu.VMEM((ntpw + 16,), jnp.int32),
            pltpu.VMEM((ntpw,), jnp.float32),
            pltpu.SemaphoreType.DMA,
        ],
        compiler_params=cp,
    )(x, grouped, rs_pad, ti, tj, w1, w2)
    return out


def kernel(x, adj, tar_ei, NCN_mode, W_xslin, b_xslin):
    n_nodes, d = x.shape
    n_edges = adj.shape[1]
    n_targets = tar_ei.shape[1]
    w1 = W_xslin[0, :d]
    w2 = W_xslin[0, d:]
    raw = _ncn_sc(x, adj, tar_ei[0], tar_ei[1], w1, w2,
                  n_nodes=n_nodes, n_edges=n_edges, n_targets=n_targets)
    mode_ok = jnp.asarray(jnp.asarray(NCN_mode) == 1, x.dtype)
    return ((raw + b_xslin[0]) * mode_ok).reshape(n_targets, 1)


# speedup vs baseline: 2.9466x; 2.9466x over previous
import functools

import jax
import jax.numpy as jnp
from jax import lax
from jax.experimental import pallas as pl
from jax.experimental.pallas import tpu as pltpu
from jax.experimental.pallas import tpu_sc as plsc

NC = 2
NS = 16
NW = NC * NS
D = 256
CAP = 2048
LCH = 128
RS_PAD = 10240


def _sread(ref, idx):
    return ref[pl.ds(idx, 16)][0]


def _sc_body(n_targets, x_hbm, dst_hbm, rs_hbm, ti_hbm, tj_hbm, w1_hbm, w2_hbm,
             out_hbm, rs_v, ib_v, jb_v, xia_v, xja_v, lia_v, lja_v, xv_v,
             w1_v, w2_v, ti_v, tj_v, out_v, red_v, sem):
    ntpw = n_targets // NW
    wid = lax.axis_index("s") * NC + lax.axis_index("c")
    base = wid * ntpw
    pltpu.sync_copy(rs_hbm, rs_v)
    pltpu.sync_copy(ti_hbm.at[pl.ds(base, ntpw)], ti_v.at[pl.ds(0, ntpw)])
    pltpu.sync_copy(tj_hbm.at[pl.ds(base, ntpw)], tj_v.at[pl.ds(0, ntpw)])
    pltpu.sync_copy(w1_hbm, w1_v)
    pltpu.sync_copy(w2_hbm, w2_v)
    iota = lax.iota(jnp.int32, 16)

    def prefetch(t, _):
        i = _sread(ti_v, t)
        j = _sread(tj_v, t)
        fbi = (_sread(rs_v, i) // 8) * 8
        fbj = (_sread(rs_v, j) // 8) * 8
        pltpu.async_copy(x_hbm.at[i], xia_v.at[pl.ds(t * D, D)], sem)
        pltpu.async_copy(x_hbm.at[j], xja_v.at[pl.ds(t * D, D)], sem)
        pltpu.async_copy(dst_hbm.at[pl.ds(fbi, LCH)],
                         lia_v.at[pl.ds(t * LCH, LCH)], sem)
        pltpu.async_copy(dst_hbm.at[pl.ds(fbj, LCH)],
                         lja_v.at[pl.ds(t * LCH, LCH)], sem)
        return 0

    lax.fori_loop(0, ntpw, prefetch, 0)

    def drain(t, _):
        pltpu.make_async_copy(x_hbm.at[0], xia_v.at[pl.ds(t * D, D)], sem).wait()
        pltpu.make_async_copy(x_hbm.at[0], xja_v.at[pl.ds(t * D, D)], sem).wait()
        pltpu.make_async_copy(dst_hbm.at[pl.ds(0, LCH)],
                              lia_v.at[pl.ds(t * LCH, LCH)], sem).wait()
        pltpu.make_async_copy(dst_hbm.at[pl.ds(0, LCH)],
                              lja_v.at[pl.ds(t * LCH, LCH)], sem).wait()
        return 0

    lax.fori_loop(0, ntpw, drain, 0)

    def _intersect(acc, jref, jbase, shj, kj, iref, ibase, shi, ki, niter):
        def per_chunk(c, acc):
            gpos = c * 16 + iota
            idx = jbase + shj + gpos
            jv = plsc.load_gather(jref, [idx])
            prev = plsc.load_gather(jref, [jnp.maximum(idx - 1, 0)])
            focc = (jv != prev) | (gpos == 0)
            lo = jnp.zeros((16,), jnp.int32)
            hi = jnp.full((16,), ki, jnp.int32)
            for _ in range(niter):
                mid = (lo + hi) // 2
                mv = plsc.load_gather(iref, [ibase + shi + mid])
                ltv = mv < jv
                lo = jnp.where(ltv, mid + 1, lo)
                hi = jnp.where(ltv, hi, mid)
            fv = plsc.load_gather(iref, [ibase + shi + lo])
            member = (gpos < kj) & (lo < ki) & (fv == jv) & focc

            def m_cond(st):
                m, _ = st
                p = plsc.all_reduce_population_count(m)
                p = p if p.ndim == 0 else p[0]
                return p > 0

            def m_body(st):
                m, a = st
                r = plsc.all_reduce_ffs(m)
                l = r if r.ndim == 0 else r[0]
                v = _sread(jref, jbase + shj + c * 16 + l)
                pltpu.sync_copy(x_hbm.at[v], xv_v)
                for cc in range(D // 16):
                    sl = pl.ds(cc * 16, 16)
                    a = a + xv_v[sl] * w2_v[sl]
                return m & (iota != l), a

            member, acc = lax.while_loop(m_cond, m_body, (member, acc))
            return acc

        return lax.fori_loop(0, (kj + 15) // 16, per_chunk, acc)

    def per_target(t, ovec):
        i = _sread(ti_v, t)
        j = _sread(tj_v, t)
        rsi = _sread(rs_v, i)
        rsj = _sread(rs_v, j)
        ki = _sread(rs_v, i + 1) - rsi
        kj = _sread(rs_v, j + 1) - rsj
        shi = rsi - (rsi // 8) * 8
        shj = rsj - (rsj // 8) * 8

        acc = jnp.zeros((16,), jnp.float32)
        for cc in range(D // 16):
            a_sl = pl.ds(t * D + cc * 16, 16)
            w_sl = pl.ds(cc * 16, 16)
            acc = acc + xia_v[a_sl] * xja_v[a_sl] * w1_v[w_sl]

        def fast(acc):
            return _intersect(acc, lja_v, t * LCH, shj, kj,
                              lia_v, t * LCH, shi, ki, 8)

        def slow(acc):
            kic = jnp.minimum(ki, CAP)
            kjc = jnp.minimum(kj, CAP)

            def cp_i(c, _):
                pltpu.sync_copy(dst_hbm.at[pl.ds(rsi - shi + c * LCH, LCH)],
                                ib_v.at[pl.ds(c * LCH, LCH)])
                return 0

            def cp_j(c, _):
                pltpu.sync_copy(dst_hbm.at[pl.ds(rsj - shj + c * LCH, LCH)],
                                jb_v.at[pl.ds(c * LCH, LCH)])
                return 0

            lax.fori_loop(0, (kic + shi + LCH - 1) // LCH, cp_i, 0)
            lax.fori_loop(0, (kjc + shj + LCH - 1) // LCH, cp_j, 0)
            return _intersect(acc, jb_v, 0, shj, kjc, ib_v, 0, shi, kic, 12)

        overflow = (shi + ki > LCH) | (shj + kj > LCH)
        acc = lax.cond(overflow, slow, fast, acc)

        for sh in (8, 4, 2, 1):
            red_v[...] = acc
            acc = acc + plsc.load_gather(red_v, [iota ^ sh])
        return jnp.where(iota == t % 16, acc, ovec)

    def per_group(g, _):
        ovec = lax.fori_loop(g * 16, g * 16 + 16, per_target,
                             jnp.zeros((16,), jnp.float32))
        out_v[pl.ds(g * 16, 16)] = ovec
        return 0

    lax.fori_loop(0, ntpw // 16, per_group, 0)
    pltpu.sync_copy(out_v, out_hbm.at[pl.ds(base, ntpw)])


@functools.partial(jax.jit, static_argnames=("n_nodes", "n_targets"))
def _ncn_sc(x, dst_pad, rs_pad, ti, tj, w1, w2, n_nodes, n_targets):
    ntpw = n_targets // NW
    mesh = plsc.VectorSubcoreMesh(core_axis_name="c", subcore_axis_name="s")
    f = pl.kernel(
        functools.partial(_sc_body, n_targets),
        out_type=jax.ShapeDtypeStruct((n_targets,), jnp.float32),
        mesh=mesh,
        scratch_types=[
            pltpu.VMEM((RS_PAD,), jnp.int32),
            pltpu.VMEM((CAP + LCH,), jnp.int32),
            pltpu.VMEM((CAP + LCH,), jnp.int32),
            pltpu.VMEM((ntpw * D + 16,), jnp.float32),
            pltpu.VMEM((ntpw * D + 16,), jnp.float32),
            pltpu.VMEM((ntpw * LCH + 16,), jnp.int32),
            pltpu.VMEM((ntpw * LCH + 16,), jnp.int32),
            pltpu.VMEM((D,), jnp.float32),
            pltpu.VMEM((D,), jnp.float32),
            pltpu.VMEM((D,), jnp.float32),
            pltpu.VMEM((ntpw + 16,), jnp.int32),
            pltpu.VMEM((ntpw + 16,), jnp.int32),
            pltpu.VMEM((ntpw,), jnp.float32),
            pltpu.VMEM((16,), jnp.float32),
            pltpu.SemaphoreType.DMA,
        ],
        compiler_params=pltpu.CompilerParams(needs_layout_passes=False),
    )
    return f(x, dst_pad, rs_pad, ti, tj, w1, w2)


def kernel(x, adj, tar_ei, NCN_mode, W_xslin, b_xslin):
    n_nodes, d = x.shape
    n_edges = adj.shape[1]
    n_targets = tar_ei.shape[1]
    w1 = W_xslin[0, :d]
    w2 = W_xslin[0, d:]
    keys = jnp.sort(adj[0] * n_nodes + adj[1])
    dst_pad = jnp.concatenate(
        [keys % n_nodes, jnp.full((LCH + 64,), n_nodes, jnp.int32)])
    counts = jnp.zeros((n_nodes,), jnp.int32).at[adj[0]].add(1)
    row_start = jnp.concatenate(
        [jnp.zeros((1,), jnp.int32), jnp.cumsum(counts, dtype=jnp.int32)])
    rs_pad = jnp.concatenate(
        [row_start, jnp.full((RS_PAD - n_nodes - 1,), n_edges, jnp.int32)])
    raw = _ncn_sc(x, dst_pad, rs_pad, tar_ei[0], tar_ei[1], w1, w2,
                  n_nodes=n_nodes, n_targets=n_targets)
    mode_ok = jnp.asarray(jnp.asarray(NCN_mode) == 1, x.dtype)
    return ((raw + b_xslin[0]) * mode_ok).reshape(n_targets, 1)
